# baseline (speedup 1.0000x reference)
import functools

import jax
import jax.numpy as jnp
import numpy as np
from jax import lax
from jax.experimental import pallas as pl
from jax.experimental.pallas import tpu as pltpu
from jax.experimental.pallas import tpu_sc as plsc

N = 262144
N_LEVELS = 16
FPL = 2
LOG2_T = 19
T = 2 ** LOG2_T
TMASK = T - 1
BASE_RES = 16
MAX_RES = 4096
PER_LEVEL_SCALE = float(np.exp((np.log(MAX_RES) - np.log(BASE_RES)) / (N_LEVELS - 1)))
SCALES = [float(np.exp2(l * np.log2(PER_LEVEL_SCALE)) * BASE_RES - 1.0)
          for l in range(N_LEVELS)]
P1 = int(np.uint32(2654435761).astype(np.int32))
P2 = 805459861

NC, NS = 2, 16
NW = NC * NS
PT = N // NW
CHUNK = 64
NCHUNK = PT // CHUNK
NIDX = N_LEVELS * 8 * CHUNK
NSTREAM = NIDX // 128


def _sc_encode_body(px_hbm, py_hbm, pz_hbm, table_hbm, feat_hbm,
                    xb, yb, zb, idxf, rowsb, wbuf, featb, sem):
    wid = lax.axis_index("s") * NC + lax.axis_index("c")
    tbase = wid * PT
    pltpu.sync_copy(px_hbm.at[pl.ds(tbase, PT)], xb)
    pltpu.sync_copy(py_hbm.at[pl.ds(tbase, PT)], yb)
    pltpu.sync_copy(pz_hbm.at[pl.ds(tbase, PT)], zb)

    lane = lax.iota(jnp.int32, 16)
    half = lane >> 1
    par = lane & 1

    def chunk_body(ch, carry):
        coff = ch * CHUNK

        def pb_body(pb, c2):
            po = coff + pb * 16
            xn = (xb[pl.ds(po, 16)] + 1.0) * 0.5
            yn = (yb[pl.ds(po, 16)] + 1.0) * 0.5
            zn = (zb[pl.ds(po, 16)] + 1.0) * 0.5
            for l in range(N_LEVELS):
                s = SCALES[l]
                ppx = xn * s + 0.5
                ppy = yn * s + 0.5
                ppz = zn * s + 0.5
                ix = ppx.astype(jnp.int32)
                iy = ppy.astype(jnp.int32)
                iz = ppz.astype(jnp.int32)
                fx = ppx - ix.astype(jnp.float32)
                fy = ppy - iy.astype(jnp.float32)
                fz = ppz - iz.astype(jnp.float32)
                wx = fx * fx * (3.0 - 2.0 * fx)
                wy = fy * fy * (3.0 - 2.0 * fy)
                wz = fz * fz * (3.0 - 2.0 * fz)
                ox = 1.0 - wx
                oy = 1.0 - wy
                oz = 1.0 - wz
                hx0 = ix
                hx1 = ix + 1
                hy0 = iy * P1
                hy1 = hy0 + P1
                hz0 = iz * P2
                hz1 = hz0 + P2
                for c in range(8):
                    hh = ((hx1 if c & 1 else hx0)
                          ^ (hy1 if c & 2 else hy0)
                          ^ (hz1 if c & 4 else hz0))
                    idx = (hh & TMASK) + l * T
                    wc = ((wx if c & 1 else ox)
                          * (wy if c & 2 else oy)
                          * (wz if c & 4 else oz))
                    e = (l * 8 + c) * CHUNK + pb * 16
                    idxf[pl.ds(e, 16)] = idx
                    wbuf[pl.ds(e, 16)] = wc
            return c2

        lax.fori_loop(0, CHUNK // 16, pb_body, 0)

        def grp_body(g, c2):
            def st_body(j, c3):
                pltpu.async_copy(table_hbm.at[idxf.at[pl.ds(j * 128, 128)]],
                                 rowsb.at[pl.ds(j * 128, 128)], sem)
                return c3

            lax.fori_loop(g * 16, g * 16 + 16, st_body, 0)

            def dr_body(j, c3):
                pltpu.make_async_copy(
                    table_hbm.at[idxf.at[pl.ds(j * 128, 128)]],
                    rowsb.at[pl.ds(j * 128, 128)], sem).wait()
                return c3

            lax.fori_loop(g * 16, g * 16 + 16, dr_body, 0)
            return c2

        lax.fori_loop(0, NSTREAM // 16, grp_body, 0)

        def pg_body(pg, c2):
            p8 = pg * 8
            for l in range(N_LEVELS):
                acc = jnp.zeros((16,), jnp.float32)
                for c in range(8):
                    e0 = (l * 8 + c) * CHUNK + p8
                    r = plsc.load_gather(rowsb, [e0 + half, par])
                    w2 = plsc.load_gather(wbuf, [e0 + half])
                    acc = acc + w2 * r
                plsc.store_scatter(featb, [p8 + half, l * 2 + par], acc)
            return c2

        lax.fori_loop(0, CHUNK // 8, pg_body, 0)

        pltpu.sync_copy(featb, feat_hbm.at[pl.ds(tbase + coff, CHUNK)])
        return carry

    lax.fori_loop(0, NCHUNK, chunk_body, 0)


def _sc_encode(px, py, pz, table_flat):
    mesh = plsc.VectorSubcoreMesh(core_axis_name="c", subcore_axis_name="s",
                                  num_cores=NC, num_subcores=NS)
    f = pl.kernel(
        _sc_encode_body,
        out_type=jax.ShapeDtypeStruct((N, 2 * N_LEVELS), jnp.float32),
        mesh=mesh,
        compiler_params=pltpu.CompilerParams(needs_layout_passes=False,
                                             use_tc_tiling_on_sc=False),
        scratch_types=[
            pltpu.VMEM((PT,), jnp.float32),
            pltpu.VMEM((PT,), jnp.float32),
            pltpu.VMEM((PT,), jnp.float32),
            pltpu.VMEM((NIDX,), jnp.int32),
            pltpu.VMEM((NIDX, FPL), jnp.float32),
            pltpu.VMEM((NIDX,), jnp.float32),
            pltpu.VMEM((CHUNK, 2 * N_LEVELS), jnp.float32),
            pltpu.SemaphoreType.DMA,
        ],
    )
    return f(px, py, pz, table_flat)


def _sh_deg4(d):
    x, y, z = d[:, 0], d[:, 1], d[:, 2]
    xx, yy, zz = x * x, y * y, z * z
    xy, yz, xz = x * y, y * z, x * z
    return jnp.stack([
        jnp.full_like(x, 0.28209479177387814),
        -0.48860251190291987 * y,
        0.48860251190291987 * z,
        -0.48860251190291987 * x,
        1.0925484305920792 * xy,
        -1.0925484305920792 * yz,
        0.94617469575755997 * zz - 0.31539156525252005,
        -1.0925484305920792 * xz,
        0.54627421529603959 * xx - 0.54627421529603959 * yy,
        0.59004358992664352 * y * (-3.0 * xx + yy),
        2.8906114426405538 * xy * z,
        0.45704579946446572 * y * (1.0 - 5.0 * zz),
        0.3731763325901154 * z * (5.0 * zz - 3.0),
        0.45704579946446572 * x * (1.0 - 5.0 * zz),
        1.4453057213202769 * z * (xx - yy),
        0.59004358992664352 * x * (-xx + 3.0 * yy),
    ], axis=-1)


def _tc_head_body(pos_ref, dir_ref, feat_ref, w1_ref, w2_ref, wr1_ref,
                  wr2_ref, out_ref):
    p = pos_ref[...]
    feat = feat_ref[...]
    tau = 10.0 * (1.0 - jnp.sqrt(jnp.sum(p * p, axis=-1)) / 0.5)
    xs = (p + 1.0) * 0.5
    sel = jnp.all((xs > 0.0) & (xs < 1.0), axis=-1)
    sig_h = jax.nn.relu(jnp.dot(feat, w1_ref[...],
                                preferred_element_type=jnp.float32))
    db = jnp.dot(sig_h, w2_ref[...], preferred_element_type=jnp.float32)
    db = db + tau[:, None] - 1.0
    dens = jax.nn.softplus(db) * sel[:, None]
    d = ((dir_ref[...] + 1.0) * 0.5) * 2.0 - 1.0
    sh = _sh_deg4(d)
    wr1 = wr1_ref[...]
    rgb_h = jax.nn.relu(
        jnp.dot(sh, wr1[:16, :], preferred_element_type=jnp.float32)
        + jnp.dot(feat, wr1[16:, :], preferred_element_type=jnp.float32))
    rgb = jax.nn.sigmoid(jnp.dot(rgb_h, wr2_ref[...],
                                 preferred_element_type=jnp.float32))
    out_ref[...] = jnp.concatenate(
        [rgb, dens, jnp.zeros((rgb.shape[0], 4), jnp.float32)], axis=1)


def _tc_head(positions, directions, feat, W_sig1, W_sig2, W_rgb1, W_rgb2):
    B = 2048
    grid = (N // B,)
    return pl.pallas_call(
        _tc_head_body,
        grid=grid,
        in_specs=[
            pl.BlockSpec((B, 3), lambda i: (i, 0)),
            pl.BlockSpec((B, 3), lambda i: (i, 0)),
            pl.BlockSpec((B, 32), lambda i: (i, 0)),
            pl.BlockSpec((32, 32), lambda i: (0, 0)),
            pl.BlockSpec((32, 1), lambda i: (0, 0)),
            pl.BlockSpec((48, 32), lambda i: (0, 0)),
            pl.BlockSpec((32, 3), lambda i: (0, 0)),
        ],
        out_specs=pl.BlockSpec((B, 8), lambda i: (i, 0)),
        out_shape=jax.ShapeDtypeStruct((N, 8), jnp.float32),
    )(positions, directions, feat, W_sig1, W_sig2, W_rgb1, W_rgb2)


def kernel(positions, directions, hash_table, W_sig1, W_sig2, W_rgb1, W_rgb2):
    pos_t = positions.T
    px, py, pz = pos_t[0], pos_t[1], pos_t[2]
    table_flat = hash_table.reshape(N_LEVELS * T, FPL)
    feat = _sc_encode(px, py, pz, table_flat)
    out = _tc_head(positions, directions, feat, W_sig1, W_sig2, W_rgb1, W_rgb2)
    return (out[:, :3], out[:, 3:4])
